# baseline (speedup 1.0000x reference)
import functools

import jax
import jax.numpy as jnp
import numpy as np
from jax import lax
from jax.experimental import pallas as pl
from jax.experimental.pallas import tpu as pltpu
from jax.experimental.pallas import tpu_sc as plsc

N_NODES = 10000
N_EDGES = 160000
F_DIM = 256
K_DIM = 64
CUTOFF = 10.0

E_PAD = 163840
N_PAD = 10240
NW = 32
EPW = E_PAD // NW
EPT = E_PAD // 16
CH = 64
NCH = EPT // CH
GRPI = 16
NGRP = NCH // GRPI
JGC = 80
NT = 5
TN = N_NODES // NT
ET = 2048
EG = E_PAD // ET

_LOG2 = 0.6931471805599453
_EXPC = float(np.exp(-CUTOFF))
_WID = float((0.5 / ((1.0 - np.exp(-CUTOFF)) / K_DIM)) ** 2)
_CSTEP = float((np.exp(-CUTOFF) - 1.0) / (K_DIM - 1))


def _act(x):
    return jnp.maximum(x, 0.0) + jnp.log1p(jnp.exp(-jnp.abs(x))) - _LOG2


def _f32(shape):
    return jax.ShapeDtypeStruct(shape, jnp.float32)


def _d2_body(r_hbm, ii_hbm, jj_hbm, d2_hbm, r_v, ii_v, jj_v, d2_v):
    c = lax.axis_index("c")
    s = lax.axis_index("s")
    w = s * 2 + c
    base = w * EPW
    pltpu.sync_copy(r_hbm, r_v)
    pltpu.sync_copy(ii_hbm.at[pl.ds(base, EPW)], ii_v)
    pltpu.sync_copy(jj_hbm.at[pl.ds(base, EPW)], jj_v)

    def batch(t, carry):
        sl = pl.ds(t * 16, 16)
        ib = ii_v[sl] * 3
        jb = jj_v[sl] * 3
        dx = plsc.load_gather(r_v, [ib]) - plsc.load_gather(r_v, [jb])
        dy = plsc.load_gather(r_v, [ib + 1]) - plsc.load_gather(r_v, [jb + 1])
        dz = plsc.load_gather(r_v, [ib + 2]) - plsc.load_gather(r_v, [jb + 2])
        d2_v[sl] = dx * dx + dy * dy + dz * dz
        return carry

    lax.fori_loop(0, EPW // 16, batch, 0)
    pltpu.sync_copy(d2_v, d2_hbm.at[pl.ds(base, EPW)])


def _d2_call(r_flat, ii, jj):
    mesh = plsc.VectorSubcoreMesh(core_axis_name="c", subcore_axis_name="s", num_cores=2, num_subcores=16)
    k = functools.partial(
        pl.kernel,
        out_type=_f32((E_PAD,)),
        mesh=mesh,
        compiler_params=pltpu.CompilerParams(needs_layout_passes=False),
        scratch_types=[
            pltpu.VMEM((3 * N_NODES,), jnp.float32),
            pltpu.VMEM((EPW,), jnp.int32),
            pltpu.VMEM((EPW,), jnp.int32),
            pltpu.VMEM((EPW,), jnp.float32),
        ],
    )(_d2_body)
    return k(r_flat, ii, jj)


def _msg_body(g0_hbm, g1_hbm, x0_hbm, x1_hbm, ii_hbm, jj_hbm, z_hbm,
              o0_hbm, o1_hbm, acc, jj_v, ii_v,
              rows0, rows1, gb0, gb1, sg0, sg1, sl0, sl1, ss0, ss1):
    c = lax.axis_index("c")
    s = lax.axis_index("s")
    rsl = pl.ds(s * (N_PAD // 16), N_PAD // 16)
    pltpu.sync_copy(z_hbm, acc.at[rsl])
    pltpu.sync_copy(jj_hbm.at[s], jj_v)
    plsc.subcore_barrier()

    rows = (rows0, rows1)
    gbs = (gb0, gb1)
    sgs = (sg0, sg1)
    sls = (sl0, sl1)
    sss = (ss0, ss1)

    def run(g_hbm, x_hbm):
        def issue(q, qq, b):
            pltpu.async_copy(x_hbm.at[jj_v.at[qq]], rows[b], sgs[b])
            pltpu.async_copy(g_hbm.at[pl.ds(s * EPT + q * CH, CH)], gbs[b], sls[b])

        def drain(q, qq, b):
            pltpu.make_async_copy(x_hbm.at[jj_v.at[qq]], rows[b], sgs[b]).wait()
            pltpu.make_async_copy(
                g_hbm.at[pl.ds(s * EPT + q * CH, CH)], gbs[b], sls[b]
            ).wait()

        def scat_wait(b):
            pltpu.make_async_copy(rows[b], acc.at[ii_v.at[0]], sss[b]).wait()

        def jgrp(j, carry0):
            pltpu.sync_copy(jj_hbm.at[s * (NCH // JGC) + j], jj_v)
            pl.when(j > 0)(functools.partial(scat_wait, 0))
            issue(j * JGC, 0, 0)

            def igrp(gi, carry1):
                pltpu.sync_copy(
                    ii_hbm.at[s * NGRP + j * (JGC // GRPI) + gi], ii_v
                )

                def pair(pp, carry):
                    for b in (0, 1):
                        lq = gi * GRPI + pp * 2 + b
                        q = j * JGC + lq
                        if b == 0:
                            pl.when(j + gi + pp > 0)(
                                functools.partial(scat_wait, 1)
                            )
                        else:
                            pl.when(lq + 1 < JGC)(
                                functools.partial(scat_wait, 0)
                            )
                        pl.when(lq + 1 < JGC)(
                            functools.partial(issue, q + 1, lq + 1, 1 - b)
                        )
                        drain(q, lq, b)

                        def rowm(r2, carry2, _b=b):
                            for dr in range(2):
                                r = r2 * 2 + dr
                                for t in range(8):
                                    sl = pl.ds(t * 16, 16)
                                    rows[_b][r, sl] = (
                                        rows[_b][r, sl] * gbs[_b][r, sl]
                                    )
                            return carry2

                        lax.fori_loop(0, CH // 2, rowm, 0)
                        pltpu.async_copy(
                            rows[b], acc.at[ii_v.at[pp * 2 + b]], sss[b], add=True
                        )
                    return carry

                lax.fori_loop(0, GRPI // 2, pair, 0)
                return carry1

            lax.fori_loop(0, JGC // GRPI, igrp, 0)
            return carry0

        lax.fori_loop(0, NCH // JGC, jgrp, 0)
        scat_wait(0)
        scat_wait(1)

    pl.when(c == 0)(lambda: run(g0_hbm, x0_hbm))
    pl.when(c == 1)(lambda: run(g1_hbm, x1_hbm))
    plsc.subcore_barrier()
    pl.when(c == 0)(lambda: pltpu.sync_copy(acc.at[rsl], o0_hbm.at[rsl]))
    pl.when(c == 1)(lambda: pltpu.sync_copy(acc.at[rsl], o1_hbm.at[rsl]))


def _msg_call(g0, g1, x0, x1, ii3, jj3, zrows):
    mesh = plsc.VectorSubcoreMesh(core_axis_name="c", subcore_axis_name="s", num_cores=2, num_subcores=16)
    k = functools.partial(
        pl.kernel,
        out_type=(_f32((N_PAD, 128)), _f32((N_PAD, 128))),
        mesh=mesh,
        compiler_params=pltpu.CompilerParams(needs_layout_passes=False),
        scratch_types=[
            pltpu.VMEM_SHARED((N_PAD, 128), jnp.float32),
            pltpu.VMEM((JGC, CH), jnp.int32),
            pltpu.VMEM((GRPI, CH), jnp.int32),
            pltpu.VMEM((CH, 128), jnp.float32),
            pltpu.VMEM((CH, 128), jnp.float32),
            pltpu.VMEM((CH, 128), jnp.float32),
            pltpu.VMEM((CH, 128), jnp.float32),
            pltpu.SemaphoreType.DMA,
            pltpu.SemaphoreType.DMA,
            pltpu.SemaphoreType.DMA,
            pltpu.SemaphoreType.DMA,
            pltpu.SemaphoreType.DMA,
            pltpu.SemaphoreType.DMA,
        ],
    )(_msg_body)
    return k(g0, g1, x0, x1, ii3, jj3, zrows)


def _emb_body(z_ref, e_ref, wj_ref, bj_ref, x_ref, x0_ref, x1_ref):
    z = z_ref[...]
    ids = lax.broadcasted_iota(jnp.int32, (TN, 128), 1)
    oh = (ids == z).astype(jnp.float32)
    x = jnp.dot(oh, e_ref[...], preferred_element_type=jnp.float32)
    x_ref[...] = x
    y = jnp.dot(_act(x), wj_ref[...], preferred_element_type=jnp.float32)
    y = y + bj_ref[...]
    x0_ref[...] = y[:, :128]
    x1_ref[...] = y[:, 128:]


_emb_call = pl.pallas_call(
    _emb_body,
    grid=(NT,),
    in_specs=[
        pl.BlockSpec((TN, 1), lambda i: (i, 0)),
        pl.BlockSpec((128, F_DIM), lambda i: (0, 0)),
        pl.BlockSpec((F_DIM, F_DIM), lambda i: (0, 0)),
        pl.BlockSpec((1, F_DIM), lambda i: (0, 0)),
    ],
    out_specs=[
        pl.BlockSpec((TN, F_DIM), lambda i: (i, 0)),
        pl.BlockSpec((TN, 128), lambda i: (i, 0)),
        pl.BlockSpec((TN, 128), lambda i: (i, 0)),
    ],
    out_shape=[_f32((N_NODES, F_DIM)), _f32((N_NODES, 128)), _f32((N_NODES, 128))],
)


def _rbf_g_body(d2_ref, k2f_ref, *out_refs):
    i = pl.program_id(0)
    d2 = d2_ref[...]
    d = jnp.sqrt(jnp.maximum(d2, 0.0))
    row = lax.broadcasted_iota(jnp.int32, (ET, 1), 0) + i * ET
    xc = d * (1.0 / CUTOFF)
    x3 = xc * xc * xc
    x4 = x3 * xc
    x5 = x4 * xc
    poly = 1.0 - 6.0 * x5 + 15.0 * x4 - 10.0 * x3
    fcut = jnp.where((row < N_EDGES) & (xc < 1.0), poly, 0.0)
    centers = 1.0 + _CSTEP * lax.broadcasted_iota(
        jnp.int32, (ET, K_DIM), 1
    ).astype(jnp.float32)
    diff = jnp.exp(-d) - centers
    rbf = fcut * jnp.exp(-_WID * (diff * diff))
    for b in range(3):
        y = jnp.dot(rbf, k2f_ref[b], preferred_element_type=jnp.float32)
        out_refs[2 * b][...] = y[:, :128]
        out_refs[2 * b + 1][...] = y[:, 128:]


_rbf_g_call = pl.pallas_call(
    _rbf_g_body,
    grid=(EG,),
    in_specs=[
        pl.BlockSpec((ET, 1), lambda i: (i, 0)),
        pl.BlockSpec((3, K_DIM, F_DIM), lambda i: (0, 0, 0)),
    ],
    out_specs=[pl.BlockSpec((ET, 128), lambda i: (i, 0)) for _ in range(6)],
    out_shape=[_f32((E_PAD, 128)) for _ in range(6)],
)


def _block_body(s0_ref, s1_ref, x_ref, wi_ref, bi_ref, w1_ref, b1_ref,
                w2_ref, b2_ref, wd_ref, bd_ref, u_ref, wo_ref,
                wjn_ref, bjn_ref, xn_ref, x0_ref, x1_ref, out_ref):
    def res(v, k):
        h = _act(v)
        h = jnp.dot(h, w1_ref[k], preferred_element_type=jnp.float32) + b1_ref[k]
        return v + jnp.dot(h, w2_ref[k], preferred_element_type=jnp.float32) + b2_ref[k]

    x = x_ref[...]
    xi = (
        jnp.dot(_act(x), wi_ref[...], preferred_element_type=jnp.float32)
        + bi_ref[...]
    )
    m = xi + jnp.concatenate([s0_ref[...], s1_ref[...]], axis=1)
    m = res(m, 0)
    m = res(m, 1)
    m = _act(m)
    xn = (
        u_ref[...] * x
        + jnp.dot(m, wd_ref[...], preferred_element_type=jnp.float32)
        + bd_ref[...]
    )
    xn = res(xn, 2)
    xn = res(xn, 3)
    xn_ref[...] = xn
    o = _act(res(xn, 4))
    out_ref[...] = jnp.dot(o, wo_ref[...], preferred_element_type=jnp.float32)
    y = jnp.dot(_act(xn), wjn_ref[...], preferred_element_type=jnp.float32)
    y = y + bjn_ref[...]
    x0_ref[...] = y[:, :128]
    x1_ref[...] = y[:, 128:]


_block_call = pl.pallas_call(
    _block_body,
    grid=(NT,),
    in_specs=[
        pl.BlockSpec((TN, 128), lambda i: (i, 0)),
        pl.BlockSpec((TN, 128), lambda i: (i, 0)),
        pl.BlockSpec((TN, F_DIM), lambda i: (i, 0)),
        pl.BlockSpec((F_DIM, F_DIM), lambda i: (0, 0)),
        pl.BlockSpec((1, F_DIM), lambda i: (0, 0)),
        pl.BlockSpec((5, F_DIM, F_DIM), lambda i: (0, 0, 0)),
        pl.BlockSpec((5, 1, F_DIM), lambda i: (0, 0, 0)),
        pl.BlockSpec((5, F_DIM, F_DIM), lambda i: (0, 0, 0)),
        pl.BlockSpec((5, 1, F_DIM), lambda i: (0, 0, 0)),
        pl.BlockSpec((F_DIM, F_DIM), lambda i: (0, 0)),
        pl.BlockSpec((1, F_DIM), lambda i: (0, 0)),
        pl.BlockSpec((1, F_DIM), lambda i: (0, 0)),
        pl.BlockSpec((F_DIM, 2), lambda i: (0, 0)),
        pl.BlockSpec((F_DIM, F_DIM), lambda i: (0, 0)),
        pl.BlockSpec((1, F_DIM), lambda i: (0, 0)),
    ],
    out_specs=[
        pl.BlockSpec((TN, F_DIM), lambda i: (i, 0)),
        pl.BlockSpec((TN, 128), lambda i: (i, 0)),
        pl.BlockSpec((TN, 128), lambda i: (i, 0)),
        pl.BlockSpec((TN, 2), lambda i: (i, 0)),
    ],
    out_shape=[
        _f32((N_NODES, F_DIM)),
        _f32((N_NODES, 128)),
        _f32((N_NODES, 128)),
        _f32((N_NODES, 2)),
    ],
)


def _final_body(o0_ref, o1_ref, o2_ref, e_ref, q_ref, nh_ref):
    i = pl.program_id(0)
    a = o0_ref[...]
    b = o1_ref[...]
    c = o2_ref[...]
    t = a + b + c
    e_ref[...] = t[:, :1]
    q_ref[...] = t[:, 1:2]
    a2 = a * a
    b2 = b * b
    c2 = c * c
    v = b2 / (b2 + a2 + 1e-7) + c2 / (c2 + b2 + 1e-7)
    p = jnp.sum(v).reshape(1, 1)

    @pl.when(i == 0)
    def _():
        nh_ref[...] = jnp.zeros((1, 1), jnp.float32)

    nh_ref[...] += p

    @pl.when(i == NT - 1)
    def _():
        nh_ref[...] *= 1.0 / (2.0 * N_NODES)


_final_call = pl.pallas_call(
    _final_body,
    grid=(NT,),
    in_specs=[pl.BlockSpec((TN, 2), lambda i: (i, 0)) for _ in range(3)],
    out_specs=[
        pl.BlockSpec((TN, 1), lambda i: (i, 0)),
        pl.BlockSpec((TN, 1), lambda i: (i, 0)),
        pl.BlockSpec((1, 1), lambda i: (0, 0)),
    ],
    out_shape=[_f32((N_NODES, 1)), _f32((N_NODES, 1)), _f32((1, 1))],
)


@jax.jit
def _run(R, params, Z, idx_i, idx_j):
    pad = jnp.zeros((E_PAD - N_EDGES,), jnp.int32)
    ii = jnp.concatenate([idx_i, pad])
    jj = jnp.concatenate([idx_j, pad])
    ii3 = ii.reshape(16 * NGRP, GRPI, CH)
    jj3 = jj.reshape(16 * (NCH // JGC), JGC, CH)

    d2 = _d2_call(R.reshape(-1), ii, jj)
    k2f_all = jnp.stack(
        [params["blocks"][b]["interaction"]["k2f"]["W"] for b in range(3)]
    )
    g = _rbf_g_call(d2.reshape(E_PAD, 1), k2f_all)

    emb_pad = jnp.zeros((128, F_DIM), jnp.float32).at[:20].set(params["atom_emb"])
    ip0 = params["blocks"][0]["interaction"]
    x, x0, x1 = _emb_call(
        Z.reshape(-1, 1), emb_pad,
        ip0["dense_j"]["W"], ip0["dense_j"]["b"].reshape(1, F_DIM),
    )

    zrows = jnp.zeros((N_PAD // 16, 128), jnp.float32)
    outs = []
    for b in range(3):
        blk = params["blocks"][b]
        ip = blk["interaction"]
        ipn = params["blocks"][(b + 1) % 3]["interaction"]
        s0, s1 = _msg_call(g[2 * b], g[2 * b + 1], x0, x1, ii3, jj3, zrows)
        rps = (
            list(ip["residuals"])
            + list(blk["atom_residuals"])
            + list(blk["output"]["residuals"])
        )
        w1 = jnp.stack([rp["dense1"]["W"] for rp in rps])
        b1 = jnp.stack([rp["dense1"]["b"].reshape(1, F_DIM) for rp in rps])
        w2 = jnp.stack([rp["dense2"]["W"] for rp in rps])
        b2 = jnp.stack([rp["dense2"]["b"].reshape(1, F_DIM) for rp in rps])
        x, x0, x1, outb = _block_call(
            s0, s1, x,
            ip["dense_i"]["W"], ip["dense_i"]["b"].reshape(1, F_DIM),
            w1, b1, w2, b2,
            ip["dense"]["W"], ip["dense"]["b"].reshape(1, F_DIM),
            ip["u"].reshape(1, F_DIM),
            blk["output"]["dense"]["W"],
            ipn["dense_j"]["W"], ipn["dense_j"]["b"].reshape(1, F_DIM),
        )
        outs.append(outb)

    e, q, nh = _final_call(*outs)
    return e.reshape(-1), q.reshape(-1), nh.reshape(())


def kernel(R, params, Z, idx_i, idx_j):
    return _run(R, params, Z, idx_i, idx_j)


# speedup vs baseline: 1.0270x; 1.0270x over previous
import functools

import jax
import jax.numpy as jnp
import numpy as np
from jax import lax
from jax.experimental import pallas as pl
from jax.experimental.pallas import tpu as pltpu
from jax.experimental.pallas import tpu_sc as plsc

N_NODES = 10000
N_EDGES = 160000
F_DIM = 256
K_DIM = 64
CUTOFF = 10.0

E_PAD = 163840
N_PAD = 10240
NW = 32
EPW = E_PAD // NW
EPT = E_PAD // 16
CH = 64
NCH = EPT // CH
GRPI = 16
NGRP = NCH // GRPI
JGC = 80
NT = 5
TN = N_NODES // NT
ET = 2048
EG = E_PAD // ET

_LOG2 = 0.6931471805599453
_EXPC = float(np.exp(-CUTOFF))
_WID = float((0.5 / ((1.0 - np.exp(-CUTOFF)) / K_DIM)) ** 2)
_CSTEP = float((np.exp(-CUTOFF) - 1.0) / (K_DIM - 1))


def _act(x):
    return jnp.maximum(x, 0.0) + jnp.log1p(jnp.exp(-jnp.abs(x))) - _LOG2


def _f32(shape):
    return jax.ShapeDtypeStruct(shape, jnp.float32)


def _d2_body(r_hbm, ii_hbm, jj_hbm, d2_hbm, r_v, ii_v, jj_v, d2_v):
    c = lax.axis_index("c")
    s = lax.axis_index("s")
    w = s * 2 + c
    base = w * EPW
    pltpu.sync_copy(r_hbm, r_v)
    pltpu.sync_copy(ii_hbm.at[pl.ds(base, EPW)], ii_v)
    pltpu.sync_copy(jj_hbm.at[pl.ds(base, EPW)], jj_v)

    def batch(t, carry):
        sl = pl.ds(t * 16, 16)
        ib = ii_v[sl] * 3
        jb = jj_v[sl] * 3
        dx = plsc.load_gather(r_v, [ib]) - plsc.load_gather(r_v, [jb])
        dy = plsc.load_gather(r_v, [ib + 1]) - plsc.load_gather(r_v, [jb + 1])
        dz = plsc.load_gather(r_v, [ib + 2]) - plsc.load_gather(r_v, [jb + 2])
        d2_v[sl] = dx * dx + dy * dy + dz * dz
        return carry

    lax.fori_loop(0, EPW // 16, batch, 0)
    pltpu.sync_copy(d2_v, d2_hbm.at[pl.ds(base, EPW)])


def _d2_call(r_flat, ii, jj):
    mesh = plsc.VectorSubcoreMesh(core_axis_name="c", subcore_axis_name="s", num_cores=2, num_subcores=16)
    k = functools.partial(
        pl.kernel,
        out_type=_f32((E_PAD,)),
        mesh=mesh,
        compiler_params=pltpu.CompilerParams(needs_layout_passes=False),
        scratch_types=[
            pltpu.VMEM((3 * N_NODES,), jnp.float32),
            pltpu.VMEM((EPW,), jnp.int32),
            pltpu.VMEM((EPW,), jnp.int32),
            pltpu.VMEM((EPW,), jnp.float32),
        ],
    )(_d2_body)
    return k(r_flat, ii, jj)


def _msg_body(g0_hbm, g1_hbm, x0_hbm, x1_hbm, ii_hbm, jj_hbm, z_hbm,
              o0_hbm, o1_hbm, acc, jj_v, ii_v,
              rows0, rows1, gb0, gb1, sg0, sg1, sl0, sl1, ss0, ss1):
    c = lax.axis_index("c")
    s = lax.axis_index("s")
    rsl = pl.ds(s * (N_PAD // 16), N_PAD // 16)
    pltpu.sync_copy(z_hbm, acc.at[rsl])
    pltpu.sync_copy(jj_hbm.at[s], jj_v)
    plsc.subcore_barrier()

    rows = (rows0, rows1)
    gbs = (gb0, gb1)
    sgs = (sg0, sg1)
    sls = (sl0, sl1)
    sss = (ss0, ss1)

    def run(g_hbm, x_hbm):
        def issue(q, qq, b):
            pltpu.async_copy(x_hbm.at[jj_v.at[qq]], rows[b], sgs[b])
            pltpu.async_copy(g_hbm.at[pl.ds(s * EPT + q * CH, CH)], gbs[b], sls[b])

        def drain(q, qq, b):
            pltpu.make_async_copy(x_hbm.at[jj_v.at[qq]], rows[b], sgs[b]).wait()
            pltpu.make_async_copy(
                g_hbm.at[pl.ds(s * EPT + q * CH, CH)], gbs[b], sls[b]
            ).wait()

        def scat_wait(b):
            pltpu.make_async_copy(rows[b], acc.at[ii_v.at[0]], sss[b]).wait()

        def jgrp(j, carry0):
            pltpu.sync_copy(jj_hbm.at[s * (NCH // JGC) + j], jj_v)
            pl.when(j > 0)(functools.partial(scat_wait, 0))
            issue(j * JGC, 0, 0)

            def igrp(gi, carry1):
                pltpu.sync_copy(
                    ii_hbm.at[s * NGRP + j * (JGC // GRPI) + gi], ii_v
                )

                def pair(pp, carry):
                    for b in (0, 1):
                        lq = gi * GRPI + pp * 2 + b
                        q = j * JGC + lq
                        if b == 0:
                            pl.when(j + gi + pp > 0)(
                                functools.partial(scat_wait, 1)
                            )
                        else:
                            pl.when(lq + 1 < JGC)(
                                functools.partial(scat_wait, 0)
                            )
                        pl.when(lq + 1 < JGC)(
                            functools.partial(issue, q + 1, lq + 1, 1 - b)
                        )
                        drain(q, lq, b)

                        def rowm(r, carry2, _b=b):
                            for t in range(8):
                                sl = pl.ds(t * 16, 16)
                                rows[_b][r, sl] = rows[_b][r, sl] * gbs[_b][r, sl]
                            return carry2

                        lax.fori_loop(0, CH, rowm, 0)
                        pltpu.async_copy(
                            rows[b], acc.at[ii_v.at[pp * 2 + b]], sss[b], add=True
                        )
                    return carry

                lax.fori_loop(0, GRPI // 2, pair, 0)
                return carry1

            lax.fori_loop(0, JGC // GRPI, igrp, 0)
            return carry0

        lax.fori_loop(0, NCH // JGC, jgrp, 0)
        scat_wait(0)
        scat_wait(1)

    pl.when(c == 0)(lambda: run(g0_hbm, x0_hbm))
    pl.when(c == 1)(lambda: run(g1_hbm, x1_hbm))
    plsc.subcore_barrier()
    pl.when(c == 0)(lambda: pltpu.sync_copy(acc.at[rsl], o0_hbm.at[rsl]))
    pl.when(c == 1)(lambda: pltpu.sync_copy(acc.at[rsl], o1_hbm.at[rsl]))


def _msg_call(g0, g1, x0, x1, ii3, jj3, zrows):
    mesh = plsc.VectorSubcoreMesh(core_axis_name="c", subcore_axis_name="s", num_cores=2, num_subcores=16)
    k = functools.partial(
        pl.kernel,
        out_type=(_f32((N_PAD, 128)), _f32((N_PAD, 128))),
        mesh=mesh,
        compiler_params=pltpu.CompilerParams(needs_layout_passes=False),
        scratch_types=[
            pltpu.VMEM_SHARED((N_PAD, 128), jnp.float32),
            pltpu.VMEM((JGC, CH), jnp.int32),
            pltpu.VMEM((GRPI, CH), jnp.int32),
            pltpu.VMEM((CH, 128), jnp.float32),
            pltpu.VMEM((CH, 128), jnp.float32),
            pltpu.VMEM((CH, 128), jnp.float32),
            pltpu.VMEM((CH, 128), jnp.float32),
            pltpu.SemaphoreType.DMA,
            pltpu.SemaphoreType.DMA,
            pltpu.SemaphoreType.DMA,
            pltpu.SemaphoreType.DMA,
            pltpu.SemaphoreType.DMA,
            pltpu.SemaphoreType.DMA,
        ],
    )(_msg_body)
    return k(g0, g1, x0, x1, ii3, jj3, zrows)


def _emb_body(z_ref, e_ref, x_ref):
    z = z_ref[...]
    ids = lax.broadcasted_iota(jnp.int32, (TN, 128), 1)
    oh = (ids == z).astype(jnp.float32)
    x_ref[...] = jnp.dot(oh, e_ref[...], preferred_element_type=jnp.float32)


_emb_call = pl.pallas_call(
    _emb_body,
    grid=(NT,),
    in_specs=[
        pl.BlockSpec((TN, 1), lambda i: (i, 0)),
        pl.BlockSpec((128, F_DIM), lambda i: (0, 0)),
    ],
    out_specs=pl.BlockSpec((TN, F_DIM), lambda i: (i, 0)),
    out_shape=_f32((N_NODES, F_DIM)),
)


def _proj_body(x_ref, wi_ref, bi_ref, wj_ref, bj_ref, xi_ref, x0_ref, x1_ref):
    xa = _act(x_ref[...])
    xi_ref[...] = (
        jnp.dot(xa, wi_ref[...], preferred_element_type=jnp.float32)
        + bi_ref[...]
    )
    y = jnp.dot(xa, wj_ref[...], preferred_element_type=jnp.float32) + bj_ref[...]
    x0_ref[...] = y[:, :128]
    x1_ref[...] = y[:, 128:]


_proj_call = pl.pallas_call(
    _proj_body,
    grid=(NT,),
    in_specs=[
        pl.BlockSpec((TN, F_DIM), lambda i: (i, 0)),
        pl.BlockSpec((F_DIM, F_DIM), lambda i: (0, 0)),
        pl.BlockSpec((1, F_DIM), lambda i: (0, 0)),
        pl.BlockSpec((F_DIM, F_DIM), lambda i: (0, 0)),
        pl.BlockSpec((1, F_DIM), lambda i: (0, 0)),
    ],
    out_specs=[
        pl.BlockSpec((TN, F_DIM), lambda i: (i, 0)),
        pl.BlockSpec((TN, 128), lambda i: (i, 0)),
        pl.BlockSpec((TN, 128), lambda i: (i, 0)),
    ],
    out_shape=[_f32((N_NODES, F_DIM)), _f32((N_NODES, 128)), _f32((N_NODES, 128))],
)


def _rbf_g_body(d2_ref, k2f_ref, *out_refs):
    i = pl.program_id(0)
    d2 = d2_ref[...]
    d = jnp.sqrt(jnp.maximum(d2, 0.0))
    row = lax.broadcasted_iota(jnp.int32, (ET, 1), 0) + i * ET
    xc = d * (1.0 / CUTOFF)
    x3 = xc * xc * xc
    x4 = x3 * xc
    x5 = x4 * xc
    poly = 1.0 - 6.0 * x5 + 15.0 * x4 - 10.0 * x3
    fcut = jnp.where((row < N_EDGES) & (xc < 1.0), poly, 0.0)
    centers = 1.0 + _CSTEP * lax.broadcasted_iota(
        jnp.int32, (ET, K_DIM), 1
    ).astype(jnp.float32)
    diff = jnp.exp(-d) - centers
    rbf = fcut * jnp.exp(-_WID * (diff * diff))
    for b in range(3):
        y = jnp.dot(rbf, k2f_ref[b], preferred_element_type=jnp.float32)
        out_refs[2 * b][...] = y[:, :128]
        out_refs[2 * b + 1][...] = y[:, 128:]


_rbf_g_call = pl.pallas_call(
    _rbf_g_body,
    grid=(EG,),
    in_specs=[
        pl.BlockSpec((ET, 1), lambda i: (i, 0)),
        pl.BlockSpec((3, K_DIM, F_DIM), lambda i: (0, 0, 0)),
    ],
    out_specs=[pl.BlockSpec((ET, 128), lambda i: (i, 0)) for _ in range(6)],
    out_shape=[_f32((E_PAD, 128)) for _ in range(6)],
)


def _block_body(xi_ref, s0_ref, s1_ref, x_ref, w1_ref, b1_ref, w2_ref, b2_ref,
                wd_ref, bd_ref, u_ref, wo_ref, xn_ref, out_ref):
    def res(v, k):
        h = _act(v)
        h = jnp.dot(h, w1_ref[k], preferred_element_type=jnp.float32) + b1_ref[k]
        return v + jnp.dot(h, w2_ref[k], preferred_element_type=jnp.float32) + b2_ref[k]

    m = xi_ref[...] + jnp.concatenate([s0_ref[...], s1_ref[...]], axis=1)
    m = res(m, 0)
    m = res(m, 1)
    m = _act(m)
    xn = (
        u_ref[...] * x_ref[...]
        + jnp.dot(m, wd_ref[...], preferred_element_type=jnp.float32)
        + bd_ref[...]
    )
    xn = res(xn, 2)
    xn = res(xn, 3)
    xn_ref[...] = xn
    o = _act(res(xn, 4))
    out_ref[...] = jnp.dot(o, wo_ref[...], preferred_element_type=jnp.float32)


_block_call = pl.pallas_call(
    _block_body,
    grid=(NT,),
    in_specs=[
        pl.BlockSpec((TN, F_DIM), lambda i: (i, 0)),
        pl.BlockSpec((TN, 128), lambda i: (i, 0)),
        pl.BlockSpec((TN, 128), lambda i: (i, 0)),
        pl.BlockSpec((TN, F_DIM), lambda i: (i, 0)),
        pl.BlockSpec((5, F_DIM, F_DIM), lambda i: (0, 0, 0)),
        pl.BlockSpec((5, 1, F_DIM), lambda i: (0, 0, 0)),
        pl.BlockSpec((5, F_DIM, F_DIM), lambda i: (0, 0, 0)),
        pl.BlockSpec((5, 1, F_DIM), lambda i: (0, 0, 0)),
        pl.BlockSpec((F_DIM, F_DIM), lambda i: (0, 0)),
        pl.BlockSpec((1, F_DIM), lambda i: (0, 0)),
        pl.BlockSpec((1, F_DIM), lambda i: (0, 0)),
        pl.BlockSpec((F_DIM, 2), lambda i: (0, 0)),
    ],
    out_specs=[
        pl.BlockSpec((TN, F_DIM), lambda i: (i, 0)),
        pl.BlockSpec((TN, 2), lambda i: (i, 0)),
    ],
    out_shape=[_f32((N_NODES, F_DIM)), _f32((N_NODES, 2))],
)


def _final_body(o0_ref, o1_ref, o2_ref, e_ref, q_ref, nh_ref):
    i = pl.program_id(0)
    a = o0_ref[...]
    b = o1_ref[...]
    c = o2_ref[...]
    t = a + b + c
    e_ref[...] = t[:, :1]
    q_ref[...] = t[:, 1:2]
    a2 = a * a
    b2 = b * b
    c2 = c * c
    v = b2 / (b2 + a2 + 1e-7) + c2 / (c2 + b2 + 1e-7)
    p = jnp.sum(v).reshape(1, 1)

    @pl.when(i == 0)
    def _():
        nh_ref[...] = jnp.zeros((1, 1), jnp.float32)

    nh_ref[...] += p

    @pl.when(i == NT - 1)
    def _():
        nh_ref[...] *= 1.0 / (2.0 * N_NODES)


_final_call = pl.pallas_call(
    _final_body,
    grid=(NT,),
    in_specs=[pl.BlockSpec((TN, 2), lambda i: (i, 0)) for _ in range(3)],
    out_specs=[
        pl.BlockSpec((TN, 1), lambda i: (i, 0)),
        pl.BlockSpec((TN, 1), lambda i: (i, 0)),
        pl.BlockSpec((1, 1), lambda i: (0, 0)),
    ],
    out_shape=[_f32((N_NODES, 1)), _f32((N_NODES, 1)), _f32((1, 1))],
)


@jax.jit
def _run(R, params, Z, idx_i, idx_j):
    pad = jnp.zeros((E_PAD - N_EDGES,), jnp.int32)
    ii = jnp.concatenate([idx_i, pad])
    jj = jnp.concatenate([idx_j, pad])
    ii3 = ii.reshape(16 * NGRP, GRPI, CH)
    jj3 = jj.reshape(16 * (NCH // JGC), JGC, CH)

    d2 = _d2_call(R.reshape(-1), ii, jj)
    k2f_all = jnp.stack(
        [params["blocks"][b]["interaction"]["k2f"]["W"] for b in range(3)]
    )
    g = _rbf_g_call(d2.reshape(E_PAD, 1), k2f_all)

    emb_pad = jnp.zeros((128, F_DIM), jnp.float32).at[:20].set(params["atom_emb"])
    x = _emb_call(Z.reshape(-1, 1), emb_pad)

    zrows = jnp.zeros((N_PAD // 16, 128), jnp.float32)
    outs = []
    for b in range(3):
        blk = params["blocks"][b]
        ip = blk["interaction"]
        xi, x0, x1 = _proj_call(
            x,
            ip["dense_i"]["W"], ip["dense_i"]["b"].reshape(1, F_DIM),
            ip["dense_j"]["W"], ip["dense_j"]["b"].reshape(1, F_DIM),
        )
        s0, s1 = _msg_call(g[2 * b], g[2 * b + 1], x0, x1, ii3, jj3, zrows)
        rps = (
            list(ip["residuals"])
            + list(blk["atom_residuals"])
            + list(blk["output"]["residuals"])
        )
        w1 = jnp.stack([rp["dense1"]["W"] for rp in rps])
        b1 = jnp.stack([rp["dense1"]["b"].reshape(1, F_DIM) for rp in rps])
        w2 = jnp.stack([rp["dense2"]["W"] for rp in rps])
        b2 = jnp.stack([rp["dense2"]["b"].reshape(1, F_DIM) for rp in rps])
        x, outb = _block_call(
            xi, s0, s1, x, w1, b1, w2, b2,
            ip["dense"]["W"], ip["dense"]["b"].reshape(1, F_DIM),
            ip["u"].reshape(1, F_DIM),
            blk["output"]["dense"]["W"],
        )
        outs.append(outb)

    e, q, nh = _final_call(*outs)
    return e.reshape(-1), q.reshape(-1), nh.reshape(())


def kernel(R, params, Z, idx_i, idx_j):
    return _run(R, params, Z, idx_i, idx_j)
